# baseline (speedup 1.0000x reference)
import functools

import jax
import jax.numpy as jnp
from jax import lax
from jax.experimental import pallas as pl
from jax.experimental.pallas import tpu as pltpu
from jax.experimental.pallas import tpu_sc as plsc

_B = 16384
_D = 64
_H = 128
_CH = 128
_NSLOT = 4


@functools.cache
def _gather_fn(B, NC, NS):
    NW = NC * NS
    b_per_w = B // NW
    n_ch = b_per_w // _CH
    mesh = plsc.VectorSubcoreMesh(core_axis_name="c", subcore_axis_name="s")

    @functools.partial(
        pl.kernel,
        out_type=(
            jax.ShapeDtypeStruct((B, 2 * _D), jnp.float32),
            jax.ShapeDtypeStruct((B, 2 * _D), jnp.float32),
        ),
        mesh=mesh,
        scratch_types=[
            pltpu.VMEM((n_ch, _CH), jnp.int32),
            pltpu.VMEM((n_ch, _CH), jnp.int32),
            pltpu.VMEM((_NSLOT, _CH, 2 * _D), jnp.float32),
            pltpu.SemaphoreType.DMA,
            pltpu.SemaphoreType.DMA,
            pltpu.SemaphoreType.DMA,
            pltpu.SemaphoreType.DMA,
        ],
    )
    def gather_k(uh_hbm, ih_hbm, ut_hbm, it_hbm, xu_out, xi_out,
                 uidx_v, iidx_v, rows_v, sem0, sem1, sem2, sem3):
        sems = [sem0, sem1, sem2, sem3]
        wid = lax.axis_index("s") * NC + lax.axis_index("c")
        base = wid * b_per_w
        pltpu.sync_copy(uh_hbm.at[pl.ds(wid * n_ch, n_ch)], uidx_v)
        pltpu.sync_copy(ih_hbm.at[pl.ds(wid * n_ch, n_ch)], iidx_v)
        jobs = ([(ut_hbm, uidx_v, xu_out, c) for c in range(n_ch)]
                + [(it_hbm, iidx_v, xi_out, c) for c in range(n_ch)])
        copies = [None] * len(jobs)

        def fire(j):
            tbl, idxv, _, c = jobs[j]
            copies[j] = pltpu.async_copy(
                tbl.at[idxv.at[c]], rows_v.at[j % _NSLOT], sems[j % _NSLOT])

        for j in range(_NSLOT):
            fire(j)
        for j in range(len(jobs)):
            _, _, out, c = jobs[j]
            copies[j].wait()
            pltpu.sync_copy(rows_v.at[j % _NSLOT],
                            out.at[pl.ds(base + c * _CH, _CH)])
            if j + _NSLOT < len(jobs):
                fire(j + _NSLOT)

    return gather_k


def _mlp_body(xu_ref, xi_ref, pu_ref, pi_ref,
              w1u_ref, w1i_ref, b1_ref, w2_ref, b2_ref, out_ref):
    ue = jnp.where(pu_ref[...] == 1, xu_ref[:, _D:], xu_ref[:, :_D])
    ie = jnp.where(pi_ref[...] == 1, xi_ref[:, _D:], xi_ref[:, :_D])
    h = (jnp.dot(ue, w1u_ref[...], preferred_element_type=jnp.float32)
         + jnp.dot(ie, w1i_ref[...], preferred_element_type=jnp.float32)
         + b1_ref[...])
    h = jnp.maximum(h, 0.0)
    out_ref[...] = (
        jnp.dot(h, w2_ref[...], preferred_element_type=jnp.float32) + b2_ref[...]
    )


@functools.cache
def _mlp_fn(B, D, H, blk):
    grid = B // blk
    return pl.pallas_call(
        _mlp_body,
        grid=(grid,),
        in_specs=[
            pl.BlockSpec((blk, 2 * D), lambda i: (i, 0)),
            pl.BlockSpec((blk, 2 * D), lambda i: (i, 0)),
            pl.BlockSpec((blk, 1), lambda i: (i, 0)),
            pl.BlockSpec((blk, 1), lambda i: (i, 0)),
            pl.BlockSpec((D, H), lambda i: (0, 0)),
            pl.BlockSpec((D, H), lambda i: (0, 0)),
            pl.BlockSpec((1, H), lambda i: (0, 0)),
            pl.BlockSpec((H, 1), lambda i: (0, 0)),
            pl.BlockSpec((1, 1), lambda i: (0, 0)),
        ],
        out_specs=pl.BlockSpec((blk, 1), lambda i: (i, 0)),
        out_shape=jax.ShapeDtypeStruct((B, 1), jnp.float32),
    )


def kernel(user, item, user_table, item_table, W1, b1, W2, b2):
    info = plsc.get_sparse_core_info()
    nrows = user_table.shape[0] // 2
    ut2 = user_table.reshape(nrows, 2 * _D)
    it2 = item_table.reshape(item_table.shape[0] // 2, 2 * _D)
    uh = (user >> 1).reshape(_B // _CH, _CH)
    ih = (item >> 1).reshape(_B // _CH, _CH)
    xu, xi = _gather_fn(_B, info.num_cores, info.num_subcores)(uh, ih, ut2, it2)
    w1u = W1[:, :_D].T
    w1i = W1[:, _D:].T
    out = _mlp_fn(_B, _D, _H, 2048)(
        xu, xi, (user & 1).reshape(_B, 1), (item & 1).reshape(_B, 1),
        w1u, w1i, b1.reshape(1, _H), W2.T, b2.reshape(1, 1))
    return out.reshape(_B)


# speedup vs baseline: 2.2336x; 2.2336x over previous
import functools

import jax
import jax.numpy as jnp
from jax import lax
from jax.experimental import pallas as pl
from jax.experimental.pallas import tpu as pltpu
from jax.experimental.pallas import tpu_sc as plsc

_B = 16384
_D = 64
_H = 128
_CH = 128


def _transpose_body(xa_ref, xb_ref, out_ref):
    out_ref[...] = jnp.concatenate([xa_ref[...].T, xb_ref[...].T], axis=1)


def _pack_geometry(n_rows, R):
    off = (n_rows // 2) // 128 * 128
    grid = -(-(n_rows - off) // R)
    return off, grid


@functools.cache
def _transpose_fn(n_rows, R):
    off, grid = _pack_geometry(n_rows, R)
    off_blocks = off // R
    assert off % R == 0
    return pl.pallas_call(
        _transpose_body,
        grid=(grid,),
        in_specs=[
            pl.BlockSpec((_D, R), lambda i: (0, i)),
            pl.BlockSpec((_D, R), lambda i, ob=off_blocks: (0, i + ob)),
        ],
        out_specs=pl.BlockSpec((R, 2 * _D), lambda i: (i, 0)),
        out_shape=jax.ShapeDtypeStruct((grid * R, 2 * _D), jnp.float32),
    )


@functools.cache
def _gather_fn(B, NC, NS, n_packed):
    NW = NC * NS
    b_per_w = B // NW
    n_ch = b_per_w // _CH
    mesh = plsc.VectorSubcoreMesh(core_axis_name="c", subcore_axis_name="s")

    @functools.partial(
        pl.kernel,
        out_type=jax.ShapeDtypeStruct((B, 2 * _D), jnp.float32),
        mesh=mesh,
        scratch_types=[
            pltpu.VMEM((n_ch, _CH), jnp.int32),
            pltpu.VMEM((n_ch, _CH, 2 * _D), jnp.float32),
            pltpu.SemaphoreType.DMA,
        ],
    )
    def gather_k(idx_hbm, tbl_hbm, x_out, idx_v, rows_v, sem):
        wid = lax.axis_index("s") * NC + lax.axis_index("c")
        base = wid * b_per_w
        pltpu.sync_copy(idx_hbm.at[pl.ds(wid * n_ch, n_ch)], idx_v)
        copies = [
            pltpu.async_copy(tbl_hbm.at[idx_v.at[c]], rows_v.at[c], sem)
            for c in range(n_ch)
        ]
        for c in range(n_ch):
            copies[c].wait()
            pltpu.sync_copy(rows_v.at[c],
                            x_out.at[pl.ds(base + c * _CH, _CH)])

    return gather_k


def _mlp_body(xu_ref, xi_ref, pu_ref, pi_ref,
              w1u_ref, w1i_ref, b1_ref, w2_ref, b2_ref, out_ref):
    ue = jnp.where(pu_ref[...] == 1, xu_ref[:, _D:], xu_ref[:, :_D])
    ie = jnp.where(pi_ref[...] == 1, xi_ref[:, _D:], xi_ref[:, :_D])
    h = (jnp.dot(ue, w1u_ref[...], preferred_element_type=jnp.float32)
         + jnp.dot(ie, w1i_ref[...], preferred_element_type=jnp.float32)
         + b1_ref[...])
    h = jnp.maximum(h, 0.0)
    out_ref[...] = (
        jnp.dot(h, w2_ref[...], preferred_element_type=jnp.float32) + b2_ref[...]
    )


@functools.cache
def _mlp_fn(B, D, H, blk):
    grid = B // blk
    return pl.pallas_call(
        _mlp_body,
        grid=(grid,),
        in_specs=[
            pl.BlockSpec((blk, 2 * D), lambda i: (i, 0)),
            pl.BlockSpec((blk, 2 * D), lambda i: (i, 0)),
            pl.BlockSpec((blk, 1), lambda i: (i, 0)),
            pl.BlockSpec((blk, 1), lambda i: (i, 0)),
            pl.BlockSpec((D, H), lambda i: (0, 0)),
            pl.BlockSpec((D, H), lambda i: (0, 0)),
            pl.BlockSpec((1, H), lambda i: (0, 0)),
            pl.BlockSpec((H, 1), lambda i: (0, 0)),
            pl.BlockSpec((1, 1), lambda i: (0, 0)),
        ],
        out_specs=pl.BlockSpec((blk, 1), lambda i: (i, 0)),
        out_shape=jax.ShapeDtypeStruct((B, 1), jnp.float32),
    )


def kernel(user, item, user_table, item_table, W1, b1, W2, b2):
    info = plsc.get_sparse_core_info()
    n = user_table.shape[0]
    R = 8064
    off, grid = _pack_geometry(n, R)
    tr = _transpose_fn(n, R)
    gather = _gather_fn(_B, info.num_cores, info.num_subcores, grid * R)
    ut_t = user_table.T
    it_t = item_table.T
    uq = jnp.where(user < off, user, user - off)
    iq = jnp.where(item < off, item, item - off)
    us = (user >= off).astype(jnp.int32)
    isel = (item >= off).astype(jnp.int32)
    u2 = tr(ut_t, ut_t)
    xu = gather(uq.reshape(_B // _CH, _CH), u2)
    i2 = tr(it_t, it_t)
    xi = gather(iq.reshape(_B // _CH, _CH), i2)
    out = _mlp_fn(_B, _D, _H, 2048)(
        xu, xi, us.reshape(_B, 1), isel.reshape(_B, 1),
        W1[:, :_D].T, W1[:, _D:].T, b1.reshape(1, _H), W2.T, b2.reshape(1, 1))
    return out.reshape(_B)


# speedup vs baseline: 3.1951x; 1.4305x over previous
import functools

import jax
import jax.numpy as jnp
from jax import lax
from jax.experimental import pallas as pl
from jax.experimental.pallas import tpu as pltpu
from jax.experimental.pallas import tpu_sc as plsc

_B = 16384
_D = 64
_H = 128
_CH = 128
_P = 4


def _round_bits(x):
    return lax.bitcast_convert_type(
        x.astype(jnp.bfloat16).astype(jnp.float32), jnp.int32)


def _transpose_body(x0_ref, x1_ref, x2_ref, x3_ref, out_ref):
    wa = _round_bits(jnp.concatenate([x0_ref[...].T, x1_ref[...].T], axis=1))
    wb = _round_bits(jnp.concatenate([x2_ref[...].T, x3_ref[...].T], axis=1))
    out_ref[...] = wa | lax.shift_right_logical(wb, 16)


def _pack_geometry(n_rows, R):
    off = (n_rows // _P) // R * R
    grid = -(-(n_rows - (_P - 1) * off) // R)
    return off, grid


@functools.cache
def _transpose_fn(n_rows, R):
    off, grid = _pack_geometry(n_rows, R)
    ob = off // R
    return pl.pallas_call(
        _transpose_body,
        grid=(grid,),
        in_specs=[
            pl.BlockSpec((_D, R), lambda i, k=k: (0, i + k * ob))
            for k in range(_P)
        ],
        out_specs=pl.BlockSpec((R, _P * _D // 2), lambda i: (i, 0)),
        out_shape=jax.ShapeDtypeStruct((grid * R, _P * _D // 2), jnp.int32),
    )


@functools.cache
def _gather_fn(B, NC, NS, n_packed):
    NW = NC * NS
    b_per_w = B // NW
    n_ch = b_per_w // _CH
    mesh = plsc.VectorSubcoreMesh(core_axis_name="c", subcore_axis_name="s")

    @functools.partial(
        pl.kernel,
        out_type=jax.ShapeDtypeStruct((B, _P * _D // 2), jnp.int32),
        mesh=mesh,
        scratch_types=[
            pltpu.VMEM((n_ch, _CH), jnp.int32),
            pltpu.VMEM((n_ch, _CH, _P * _D // 2), jnp.int32),
            pltpu.SemaphoreType.DMA,
        ],
    )
    def gather_k(idx_hbm, tbl_hbm, x_out, idx_v, rows_v, sem):
        wid = lax.axis_index("s") * NC + lax.axis_index("c")
        base = wid * b_per_w
        pltpu.sync_copy(idx_hbm.at[pl.ds(wid * n_ch, n_ch)], idx_v)
        copies = [
            pltpu.async_copy(tbl_hbm.at[idx_v.at[c]], rows_v.at[c], sem)
            for c in range(n_ch)
        ]
        for c in range(n_ch):
            copies[c].wait()
            pltpu.sync_copy(rows_v.at[c],
                            x_out.at[pl.ds(base + c * _CH, _CH)])

    return gather_k


def _select_quarter(x_ref, sub):
    w = x_ref[...]
    hi = lax.bitcast_convert_type(w & jnp.int32(-65536), jnp.float32)
    lo = lax.bitcast_convert_type(w << 16, jnp.float32)
    a = jnp.where(sub == 0, hi[:, :_D], hi[:, _D:])
    b = jnp.where(sub == 2, lo[:, :_D], lo[:, _D:])
    return jnp.where(sub <= 1, a, b)


def _mlp_body(xu_ref, xi_ref, pu_ref, pi_ref,
              w1u_ref, w1i_ref, b1_ref, w2_ref, b2_ref, out_ref):
    ue = _select_quarter(xu_ref, pu_ref[...])
    ie = _select_quarter(xi_ref, pi_ref[...])
    h = (jnp.dot(ue, w1u_ref[...], preferred_element_type=jnp.float32)
         + jnp.dot(ie, w1i_ref[...], preferred_element_type=jnp.float32)
         + b1_ref[...])
    h = jnp.maximum(h, 0.0)
    out_ref[...] = (
        jnp.dot(h, w2_ref[...], preferred_element_type=jnp.float32) + b2_ref[...]
    )


@functools.cache
def _mlp_fn(B, D, H, blk):
    grid = B // blk
    return pl.pallas_call(
        _mlp_body,
        grid=(grid,),
        in_specs=[
            pl.BlockSpec((blk, _P * D // 2), lambda i: (i, 0)),
            pl.BlockSpec((blk, _P * D // 2), lambda i: (i, 0)),
            pl.BlockSpec((blk, 1), lambda i: (i, 0)),
            pl.BlockSpec((blk, 1), lambda i: (i, 0)),
            pl.BlockSpec((D, H), lambda i: (0, 0)),
            pl.BlockSpec((D, H), lambda i: (0, 0)),
            pl.BlockSpec((1, H), lambda i: (0, 0)),
            pl.BlockSpec((H, 1), lambda i: (0, 0)),
            pl.BlockSpec((1, 1), lambda i: (0, 0)),
        ],
        out_specs=pl.BlockSpec((blk, 1), lambda i: (i, 0)),
        out_shape=jax.ShapeDtypeStruct((B, 1), jnp.float32),
    )


def kernel(user, item, user_table, item_table, W1, b1, W2, b2):
    info = plsc.get_sparse_core_info()
    n = user_table.shape[0]
    R = 8064
    off, grid = _pack_geometry(n, R)
    tr = _transpose_fn(n, R)
    gather = _gather_fn(_B, info.num_cores, info.num_subcores, grid * R)

    def fold(i):
        sub = ((i >= off).astype(jnp.int32) + (i >= 2 * off).astype(jnp.int32)
               + (i >= 3 * off).astype(jnp.int32))
        return i - sub * off, sub

    uq, us = fold(user)
    iq, isel = fold(item)
    u2 = tr(user_table.T, user_table.T, user_table.T, user_table.T)
    xu = gather(uq.reshape(_B // _CH, _CH), u2)
    i2 = tr(item_table.T, item_table.T, item_table.T, item_table.T)
    xi = gather(iq.reshape(_B // _CH, _CH), i2)
    out = _mlp_fn(_B, _D, _H, 2048)(
        xu, xi, us.reshape(_B, 1), isel.reshape(_B, 1),
        W1[:, :_D].T, W1[:, _D:].T, b1.reshape(1, _H), W2.T, b2.reshape(1, 1))
    return out.reshape(_B)


# speedup vs baseline: 3.2340x; 1.0122x over previous
import functools

import jax
import jax.numpy as jnp
from jax import lax
from jax.experimental import pallas as pl
from jax.experimental.pallas import tpu as pltpu
from jax.experimental.pallas import tpu_sc as plsc

_B = 16384
_D = 64
_H = 128
_CH = 128
_P = 4


def _round_bits(x):
    return lax.bitcast_convert_type(
        x.astype(jnp.bfloat16).astype(jnp.float32), jnp.int32)


def _transpose_body(x0_ref, x1_ref, x2_ref, x3_ref, out_ref):
    wa = _round_bits(jnp.concatenate([x0_ref[...].T, x1_ref[...].T], axis=1))
    wb = _round_bits(jnp.concatenate([x2_ref[...].T, x3_ref[...].T], axis=1))
    out_ref[...] = wa | lax.shift_right_logical(wb, 16)


def _pack_geometry(n_rows, R):
    off = (n_rows // _P) // R * R
    grid = -(-(n_rows - (_P - 1) * off) // R)
    return off, grid


@functools.cache
def _transpose_fn(n_rows, R):
    off, grid = _pack_geometry(n_rows, R)
    ob = off // R
    return pl.pallas_call(
        _transpose_body,
        grid=(grid,),
        in_specs=[
            pl.BlockSpec((_D, R), lambda i, k=k: (0, i + k * ob))
            for k in range(_P)
        ],
        out_specs=pl.BlockSpec((R, _P * _D // 2), lambda i: (i, 0)),
        out_shape=jax.ShapeDtypeStruct((grid * R, _P * _D // 2), jnp.int32),
    )


@functools.cache
def _gather_fn(B, NC, NS, n_packed):
    NW = NC * NS
    b_per_w = B // NW
    n_ch = b_per_w // _CH
    mesh = plsc.VectorSubcoreMesh(core_axis_name="c", subcore_axis_name="s")

    @functools.partial(
        pl.kernel,
        out_type=jax.ShapeDtypeStruct((B, _P * _D // 2), jnp.int32),
        mesh=mesh,
        scratch_types=[
            pltpu.VMEM((n_ch, _CH), jnp.int32),
            pltpu.VMEM((n_ch, _CH, _P * _D // 2), jnp.int32),
            pltpu.SemaphoreType.DMA,
        ],
    )
    def gather_k(idx_hbm, tbl_hbm, x_out, idx_v, rows_v, sem):
        wid = lax.axis_index("s") * NC + lax.axis_index("c")
        base = wid * b_per_w
        pltpu.sync_copy(idx_hbm.at[pl.ds(wid * n_ch, n_ch)], idx_v)
        copies = [
            pltpu.async_copy(tbl_hbm.at[idx_v.at[c]], rows_v.at[c], sem)
            for c in range(n_ch)
        ]
        for c in range(n_ch):
            copies[c].wait()
            pltpu.sync_copy(rows_v.at[c],
                            x_out.at[pl.ds(base + c * _CH, _CH)])

    return gather_k


def _select_quarter(x_ref, sub):
    w = x_ref[...]
    hi = lax.bitcast_convert_type(w & jnp.int32(-65536), jnp.float32)
    lo = lax.bitcast_convert_type(w << 16, jnp.float32)
    a = jnp.where(sub == 0, hi[:, :_D], hi[:, _D:])
    b = jnp.where(sub == 2, lo[:, :_D], lo[:, _D:])
    return jnp.where(sub <= 1, a, b)


def _mlp_body(xu_ref, xi_ref, pu_ref, pi_ref,
              w1u_ref, w1i_ref, b1_ref, w2_ref, b2_ref, out_ref):
    ue = _select_quarter(xu_ref, pu_ref[...])
    ie = _select_quarter(xi_ref, pi_ref[...])
    h = (jnp.dot(ue, w1u_ref[...], preferred_element_type=jnp.float32)
         + jnp.dot(ie, w1i_ref[...], preferred_element_type=jnp.float32)
         + b1_ref[...])
    h = jnp.maximum(h, 0.0)
    out_ref[...] = (
        jnp.dot(h, w2_ref[...], preferred_element_type=jnp.float32) + b2_ref[...]
    )


@functools.cache
def _mlp_fn(B, D, H, blk):
    grid = B // blk
    return pl.pallas_call(
        _mlp_body,
        grid=(grid,),
        in_specs=[
            pl.BlockSpec((blk, _P * D // 2), lambda i: (i, 0)),
            pl.BlockSpec((blk, _P * D // 2), lambda i: (i, 0)),
            pl.BlockSpec((blk, 1), lambda i: (i, 0)),
            pl.BlockSpec((blk, 1), lambda i: (i, 0)),
            pl.BlockSpec((D, H), lambda i: (0, 0)),
            pl.BlockSpec((D, H), lambda i: (0, 0)),
            pl.BlockSpec((1, H), lambda i: (0, 0)),
            pl.BlockSpec((H, 1), lambda i: (0, 0)),
            pl.BlockSpec((1, 1), lambda i: (0, 0)),
        ],
        out_specs=pl.BlockSpec((blk, 1), lambda i: (i, 0)),
        out_shape=jax.ShapeDtypeStruct((B, 1), jnp.float32),
    )


def kernel(user, item, user_table, item_table, W1, b1, W2, b2):
    info = plsc.get_sparse_core_info()
    n = user_table.shape[0]
    R = 11904
    off, grid = _pack_geometry(n, R)
    tr = _transpose_fn(n, R)
    gather = _gather_fn(_B, info.num_cores, info.num_subcores, grid * R)

    def fold(i):
        sub = ((i >= off).astype(jnp.int32) + (i >= 2 * off).astype(jnp.int32)
               + (i >= 3 * off).astype(jnp.int32))
        return i - sub * off, sub

    uq, us = fold(user)
    iq, isel = fold(item)
    u2 = tr(user_table.T, user_table.T, user_table.T, user_table.T)
    xu = gather(uq.reshape(_B // _CH, _CH), u2)
    i2 = tr(item_table.T, item_table.T, item_table.T, item_table.T)
    xi = gather(iq.reshape(_B // _CH, _CH), i2)
    out = _mlp_fn(_B, _D, _H, 2048)(
        xu, xi, us.reshape(_B, 1), isel.reshape(_B, 1),
        W1[:, :_D].T, W1[:, _D:].T, b1.reshape(1, _H), W2.T, b2.reshape(1, 1))
    return out.reshape(_B)
